# baseline (speedup 1.0000x reference)
import functools

import jax
import jax.numpy as jnp
from jax import lax
from jax.experimental import pallas as pl
from jax.experimental.pallas import tpu as pltpu
from jax.experimental.pallas import tpu_sc as plsc

NC = 2
NS = 16
L = 16
NW = NC * NS

C = 8
NBUF = 6


def _body(x_hbm, ids_hbm, tab_hbm, out_hbm, idx_v, buf_r, buf_a,
          gsem, isem, osem, *, pw, nch, h):
    cid = lax.axis_index("c")
    sid = lax.axis_index("s")
    wid = sid * NC + cid
    base = wid * pw

    pltpu.sync_copy(ids_hbm.at[wid], idx_v)

    def gather_copy(ch, b):
        return pltpu.make_async_copy(
            tab_hbm.at[idx_v.at[ch]], buf_r.at[b], gsem.at[b])

    def input_copy(ch, b):
        return pltpu.make_async_copy(
            x_hbm.at[pl.ds(base + ch * C, C)], buf_a.at[b], isem.at[b])

    def out_copy(ch, b):
        return pltpu.make_async_copy(
            buf_a.at[b], out_hbm.at[pl.ds(base + ch * C, C)], osem.at[b])

    for k in range(NBUF - 1):
        gather_copy(k, k).start()
        input_copy(k, k).start()

    def step(ch, carry):
        b = lax.rem(ch, NBUF)
        pf = ch + (NBUF - 1)
        pb = lax.rem(pf, NBUF)

        @pl.when(pf < nch)
        def _():
            gather_copy(pf, pb).start()

        gather_copy(ch, b).wait()
        input_copy(ch, b).wait()

        def row(r, c2):
            for j in range(h // L):
                s = pl.ds(j * L, L)
                plsc.addupdate(buf_a.at[b, r, s], buf_r[b, r, s])
            return c2
        lax.fori_loop(0, C, row, 0)

        out_copy(ch, b).start()

        @pl.when(pf < nch)
        def _():
            @pl.when(ch >= 1)
            def _():
                out_copy(0, pb).wait()
            input_copy(pf, pb).start()
        return carry

    lax.fori_loop(0, nch, step, 0)

    for b in range(NBUF):
        out_copy(0, b).wait()


def kernel(input_ids, position_ids, pos_table):
    bsz, seq, h = input_ids.shape
    tok = bsz * seq
    pw = tok // NW
    nch = pw // C

    x = input_ids.reshape(tok, h)
    ids = position_ids.reshape(NW, nch, C).astype(jnp.int32)

    mesh = plsc.VectorSubcoreMesh(
        core_axis_name="c", subcore_axis_name="s",
        num_cores=NC, num_subcores=NS)

    run = pl.kernel(
        functools.partial(_body, pw=pw, nch=nch, h=h),
        out_type=jax.ShapeDtypeStruct((tok, h), jnp.float32),
        mesh=mesh,
        scratch_types=[
            pltpu.VMEM((nch, C), jnp.int32),
            pltpu.VMEM((NBUF, C, h), jnp.float32),
            pltpu.VMEM((NBUF, C, h), jnp.float32),
            pltpu.SemaphoreType.DMA((NBUF,)),
            pltpu.SemaphoreType.DMA((NBUF,)),
            pltpu.SemaphoreType.DMA((NBUF,)),
        ],
    )
    out = run(x, ids, pos_table)
    return out.reshape(bsz, seq, h)


# speedup vs baseline: 1.7744x; 1.7744x over previous
import functools

import jax
import jax.numpy as jnp
from jax import lax
from jax.experimental import pallas as pl
from jax.experimental.pallas import tpu as pltpu
from jax.experimental.pallas import tpu_sc as plsc

NC = 2
NS = 16
L = 16
NW = NC * NS

C = 16
NBUF = 3


def _body(x_hbm, ids_hbm, tab_hbm, out_hbm, idx_v, buf_r, buf_a,
          gsem, isem, osem, *, pw, nch, h):
    cid = lax.axis_index("c")
    sid = lax.axis_index("s")
    wid = sid * NC + cid
    base = wid * pw

    pltpu.sync_copy(ids_hbm.at[wid], idx_v)

    def gather_copy(ch, b):
        return pltpu.make_async_copy(
            tab_hbm.at[idx_v.at[ch]], buf_r.at[b], gsem.at[b])

    def input_copy(ch, b):
        return pltpu.make_async_copy(
            x_hbm.at[pl.ds(base + ch * C, C)], buf_a.at[b], isem.at[b])

    def out_copy(ch, b):
        return pltpu.make_async_copy(
            buf_a.at[b], out_hbm.at[pl.ds(base + ch * C, C)], osem.at[b])

    def accumulate(b):
        def row(r, c2):
            for j in range(h // L):
                s = pl.ds(j * L, L)
                plsc.addupdate(buf_a.at[b, r, s], buf_r[b, r, s])
            return c2
        lax.fori_loop(0, C, row, 0)

    for k in range(NBUF - 1):
        gather_copy(k, k).start()
        input_copy(k, k).start()

    gather_copy(2, 2).start()
    gather_copy(0, 0).wait()
    input_copy(0, 0).wait()
    accumulate(0)
    out_copy(0, 0).start()
    input_copy(2, 2).start()

    def step(ch, carry):
        b = lax.rem(ch, NBUF)
        pf = ch + (NBUF - 1)
        pb = lax.rem(pf, NBUF)

        gather_copy(pf, pb).start()
        gather_copy(ch, b).wait()
        input_copy(ch, b).wait()
        accumulate(b)
        out_copy(ch, b).start()
        out_copy(0, pb).wait()
        input_copy(pf, pb).start()
        return carry

    lax.fori_loop(1, nch - NBUF + 1, step, 0)

    def tail(ch, carry):
        b = lax.rem(ch, NBUF)
        gather_copy(ch, b).wait()
        input_copy(ch, b).wait()
        accumulate(b)
        out_copy(ch, b).start()
        return carry

    lax.fori_loop(nch - NBUF + 1, nch, tail, 0)

    for b in range(NBUF):
        out_copy(0, b).wait()


def kernel(input_ids, position_ids, pos_table):
    bsz, seq, h = input_ids.shape
    tok = bsz * seq
    pw = tok // NW
    nch = pw // C

    x = input_ids.reshape(tok, h)
    ids = position_ids.reshape(NW, nch, C).astype(jnp.int32)

    mesh = plsc.VectorSubcoreMesh(
        core_axis_name="c", subcore_axis_name="s",
        num_cores=NC, num_subcores=NS)

    run = pl.kernel(
        functools.partial(_body, pw=pw, nch=nch, h=h),
        out_type=jax.ShapeDtypeStruct((tok, h), jnp.float32),
        mesh=mesh,
        scratch_types=[
            pltpu.VMEM((nch, C), jnp.int32),
            pltpu.VMEM((NBUF, C, h), jnp.float32),
            pltpu.VMEM((NBUF, C, h), jnp.float32),
            pltpu.SemaphoreType.DMA((NBUF,)),
            pltpu.SemaphoreType.DMA((NBUF,)),
            pltpu.SemaphoreType.DMA((NBUF,)),
        ],
    )
    out = run(x, ids, pos_table)
    return out.reshape(bsz, seq, h)


# speedup vs baseline: 1.7886x; 1.0080x over previous
import functools

import jax
import jax.numpy as jnp
from jax import lax
from jax.experimental import pallas as pl
from jax.experimental.pallas import tpu as pltpu
from jax.experimental.pallas import tpu_sc as plsc

NC = 2
NS = 16
L = 16
NW = NC * NS

C = 16
NBUF = 3


def _body(x_hbm, ids_hbm, tab_hbm, out_hbm, idx_v, buf_r, buf_a,
          gsem, isem, osem, *, pw, nch, h):
    cid = lax.axis_index("c")
    sid = lax.axis_index("s")
    wid = sid * NC + cid
    base = wid * pw

    pltpu.sync_copy(ids_hbm.at[wid], idx_v)

    def gather_copy(ch, b):
        return pltpu.make_async_copy(
            tab_hbm.at[idx_v.at[ch]], buf_r.at[b], gsem.at[b])

    def input_copy(ch, b):
        return pltpu.make_async_copy(
            x_hbm.at[pl.ds(base + ch * C, C)], buf_a.at[b], isem.at[b])

    def out_copy(ch, b):
        return pltpu.make_async_copy(
            buf_a.at[b], out_hbm.at[pl.ds(base + ch * C, C)], osem.at[b])

    for b in range(NBUF - 1):
        gather_copy(b, b).start()
        input_copy(b, b).start()

    def step(ch, carry):
        b = lax.rem(ch, NBUF)
        pf = ch + (NBUF - 1)
        pb = lax.rem(pf, NBUF)

        @pl.when(pf < nch)
        def _():
            gather_copy(pf, pb).start()

        gather_copy(ch, b).wait()
        input_copy(ch, b).wait()

        def row(r, c2):
            for j in range(h // L):
                s = pl.ds(j * L, L)
                plsc.addupdate(buf_a.at[b, r, s], buf_r[b, r, s])
            return c2
        lax.fori_loop(0, C, row, 0)

        out_copy(ch, b).start()

        @pl.when(pf < nch)
        def _():
            @pl.when(ch >= 1)
            def _():
                out_copy(0, pb).wait()
            input_copy(pf, pb).start()
        return carry

    lax.fori_loop(0, nch, step, 0)

    for b in range(NBUF):
        out_copy(0, b).wait()


def kernel(input_ids, position_ids, pos_table):
    bsz, seq, h = input_ids.shape
    tok = bsz * seq
    pw = tok // NW
    nch = pw // C

    x = input_ids.reshape(tok, h)
    ids = position_ids.reshape(NW, nch, C).astype(jnp.int32)

    mesh = plsc.VectorSubcoreMesh(
        core_axis_name="c", subcore_axis_name="s",
        num_cores=NC, num_subcores=NS)

    run = pl.kernel(
        functools.partial(_body, pw=pw, nch=nch, h=h),
        out_type=jax.ShapeDtypeStruct((tok, h), jnp.float32),
        mesh=mesh,
        scratch_types=[
            pltpu.VMEM((nch, C), jnp.int32),
            pltpu.VMEM((NBUF, C, h), jnp.float32),
            pltpu.VMEM((NBUF, C, h), jnp.float32),
            pltpu.SemaphoreType.DMA((NBUF,)),
            pltpu.SemaphoreType.DMA((NBUF,)),
            pltpu.SemaphoreType.DMA((NBUF,)),
        ],
    )
    out = run(x, ids, pos_table)
    return out.reshape(bsz, seq, h)
